# baseline (device time: 17515 ns/iter reference)
import jax
import jax.numpy as jnp
from jax import lax
from jax.experimental import pallas as pl
from jax.experimental.pallas import tpu as pltpu

N_DEV = 8
T = 256
D = 512
V_LOCAL = 4096
NC = 4
VC = V_LOCAL // NC


def kernel(x, W, labels):
    def body(
        x_hbm,
        w_hbm,
        labels_hbm,
        out_ref,
        x_ref,
        w_ref,
        labels_ref,
        comm_ref,
        in_sems,
        w_sems,
        send_sems,
        recv_sems,
    ):
        my = lax.axis_index("i")

        barrier = pltpu.get_barrier_semaphore()
        for d in range(1, N_DEV):
            pl.semaphore_signal(
                barrier,
                inc=1,
                device_id=((my + d) % N_DEV,),
                device_id_type=pl.DeviceIdType.MESH,
            )

        x_dma = pltpu.make_async_copy(x_hbm, x_ref, in_sems.at[0])
        l_dma = pltpu.make_async_copy(labels_hbm, labels_ref, in_sems.at[1])
        x_dma.start()
        l_dma.start()

        def w_dma(k):
            return pltpu.make_async_copy(
                w_hbm.at[:, pl.ds(k * VC, VC)],
                w_ref.at[k % 2],
                w_sems.at[k % 2],
            )

        w_dmas = [w_dma(k) for k in range(NC)]
        w_dmas[0].start()
        w_dmas[1].start()

        x_dma.wait()
        l_dma.wait()
        xb = x_ref[:, :].astype(jnp.bfloat16)
        labels_row = labels_ref[:].reshape(1, T)

        s = jnp.zeros((1, T), jnp.float32)
        ll = jnp.zeros((1, T), jnp.float32)
        for k in range(NC):
            w_dmas[k].wait()
            wb = w_ref[k % 2, :, :].astype(jnp.bfloat16)
            if k + 2 < NC:
                w_dmas[k + 2].start()
            logits_t = lax.dot_general(
                wb,
                xb,
                (((0,), (1,)), ((), ())),
                preferred_element_type=jnp.float32,
            )
            s = s + jnp.sum(jnp.exp(logits_t), axis=0, keepdims=True)
            vids = (
                lax.broadcasted_iota(jnp.int32, (VC, T), 0)
                + my * V_LOCAL
                + k * VC
            )
            mask = vids == labels_row
            ll = ll + jnp.sum(
                jnp.where(mask, logits_t, 0.0), axis=0, keepdims=True
            )

        comm_ref[0, :, :] = jnp.concatenate([s, ll], axis=0)

        pl.semaphore_wait(barrier, N_DEV - 1)

        sends = []
        for d in (4, 3, 5, 2, 6, 1, 7):
            rdma = pltpu.make_async_remote_copy(
                src_ref=comm_ref.at[0],
                dst_ref=comm_ref.at[d],
                send_sem=send_sems.at[d],
                recv_sem=recv_sems.at[d],
                device_id=((my + d) % N_DEV,),
                device_id_type=pl.DeviceIdType.MESH,
            )
            rdma.start()
            sends.append(rdma)

        for rdma in sends:
            rdma.wait_recv()
        for rdma in sends:
            rdma.wait_send()

        c = comm_ref[:, :, :]
        s_g = jnp.sum(c[:, 0:1, :], axis=0)
        ll_g = jnp.sum(c[:, 1:2, :], axis=0)
        out_ref[:] = (jnp.log(s_g) - ll_g).reshape(T)

    return pl.pallas_call(
        body,
        out_shape=jax.ShapeDtypeStruct((T,), jnp.float32),
        in_specs=[
            pl.BlockSpec(memory_space=pltpu.MemorySpace.HBM),
            pl.BlockSpec(memory_space=pltpu.MemorySpace.HBM),
            pl.BlockSpec(memory_space=pltpu.MemorySpace.HBM),
        ],
        out_specs=pl.BlockSpec(memory_space=pltpu.VMEM),
        scratch_shapes=[
            pltpu.VMEM((T, D), jnp.float32),
            pltpu.VMEM((2, D, VC), jnp.float32),
            pltpu.VMEM((T,), jnp.int32),
            pltpu.VMEM((N_DEV, 2, T), jnp.float32),
            pltpu.SemaphoreType.DMA((2,)),
            pltpu.SemaphoreType.DMA((2,)),
            pltpu.SemaphoreType.DMA((N_DEV,)),
            pltpu.SemaphoreType.DMA((N_DEV,)),
        ],
        compiler_params=pltpu.CompilerParams(collective_id=0),
    )(x, W, labels)


# device time: 14274 ns/iter; 1.2271x vs baseline; 1.2271x over previous
import jax
import jax.numpy as jnp
from jax import lax
from jax.experimental import pallas as pl
from jax.experimental.pallas import tpu as pltpu

N_DEV = 8
T = 256
D = 512
V_LOCAL = 4096
NW = 8


def kernel(x, W, labels):
    def body(
        x_hbm,
        w_hbm,
        labels_hbm,
        out_ref,
        x_ref,
        w_ref,
        labels_ref,
        comm_ref,
        in_sems,
        w_sems,
        send_sems,
        recv_sems,
    ):
        my = lax.axis_index("i")

        barrier = pltpu.get_barrier_semaphore()
        for d in range(1, N_DEV):
            pl.semaphore_signal(
                barrier,
                inc=1,
                device_id=((my + d) % N_DEV,),
                device_id_type=pl.DeviceIdType.MESH,
            )

        x_dma = pltpu.make_async_copy(x_hbm, x_ref, in_sems.at[0])
        l_dma = pltpu.make_async_copy(labels_hbm, labels_ref, in_sems.at[1])
        x_dma.start()
        l_dma.start()

        RB = D // NW
        w_dmas = [
            pltpu.make_async_copy(
                w_hbm.at[pl.ds(j * RB, RB), :],
                w_ref.at[pl.ds(j * RB, RB), :],
                w_sems.at[j],
            )
            for j in range(NW)
        ]
        for dma in w_dmas:
            dma.start()

        x_dma.wait()
        l_dma.wait()
        xb = x_ref[:, :].astype(jnp.bfloat16)
        labels_row = labels_ref[:].reshape(1, T)
        for dma in w_dmas:
            dma.wait()

        wb = w_ref[:, :].astype(jnp.bfloat16)
        logits_t = lax.dot_general(
            wb,
            xb,
            (((0,), (1,)), ((), ())),
            preferred_element_type=jnp.float32,
        )
        s = jnp.sum(jnp.exp(logits_t), axis=0, keepdims=True)
        vids = lax.broadcasted_iota(jnp.int32, (V_LOCAL, T), 0) + my * V_LOCAL
        mask = vids == labels_row
        ll = jnp.sum(jnp.where(mask, logits_t, 0.0), axis=0, keepdims=True)

        comm_ref[0, :, :] = jnp.concatenate([s, ll], axis=0)

        pl.semaphore_wait(barrier, N_DEV - 1)

        sends = []
        for d in (4, 3, 5, 2, 6, 1, 7):
            rdma = pltpu.make_async_remote_copy(
                src_ref=comm_ref.at[0],
                dst_ref=comm_ref.at[d],
                send_sem=send_sems.at[d],
                recv_sem=recv_sems.at[d],
                device_id=((my + d) % N_DEV,),
                device_id_type=pl.DeviceIdType.MESH,
            )
            rdma.start()
            sends.append(rdma)

        for rdma in sends:
            rdma.wait_recv()
        for rdma in sends:
            rdma.wait_send()

        c = comm_ref[:, :, :]
        s_g = jnp.sum(c[:, 0:1, :], axis=0)
        ll_g = jnp.sum(c[:, 1:2, :], axis=0)
        out_ref[:] = (jnp.log(s_g) - ll_g).reshape(T)

    return pl.pallas_call(
        body,
        out_shape=jax.ShapeDtypeStruct((T,), jnp.float32),
        in_specs=[
            pl.BlockSpec(memory_space=pltpu.MemorySpace.HBM),
            pl.BlockSpec(memory_space=pltpu.MemorySpace.HBM),
            pl.BlockSpec(memory_space=pltpu.MemorySpace.HBM),
        ],
        out_specs=pl.BlockSpec(memory_space=pltpu.VMEM),
        scratch_shapes=[
            pltpu.VMEM((T, D), jnp.float32),
            pltpu.VMEM((D, V_LOCAL), jnp.float32),
            pltpu.VMEM((T,), jnp.int32),
            pltpu.VMEM((N_DEV, 2, T), jnp.float32),
            pltpu.SemaphoreType.DMA((2,)),
            pltpu.SemaphoreType.DMA((NW,)),
            pltpu.SemaphoreType.DMA((N_DEV,)),
            pltpu.SemaphoreType.DMA((N_DEV,)),
        ],
        compiler_params=pltpu.CompilerParams(collective_id=0),
    )(x, W, labels)


# device time: 13998 ns/iter; 1.2513x vs baseline; 1.0197x over previous
import jax
import jax.numpy as jnp
from jax import lax
from jax.experimental import pallas as pl
from jax.experimental.pallas import tpu as pltpu

N_DEV = 8
T = 256
D = 512
V_LOCAL = 4096


def kernel(x, W, labels):
    def body(
        x_hbm,
        w_ref,
        labels_hbm,
        out_ref,
        x_ref,
        labels_ref,
        comm_ref,
        in_sems,
        send_sems,
        recv_sems,
    ):
        my = lax.axis_index("i")

        barrier = pltpu.get_barrier_semaphore()
        for d in range(1, N_DEV):
            pl.semaphore_signal(
                barrier,
                inc=1,
                device_id=((my + d) % N_DEV,),
                device_id_type=pl.DeviceIdType.MESH,
            )

        x_dma = pltpu.make_async_copy(x_hbm, x_ref, in_sems.at[0])
        l_dma = pltpu.make_async_copy(labels_hbm, labels_ref, in_sems.at[1])
        x_dma.start()
        l_dma.start()
        x_dma.wait()

        xb = x_ref[:, :].astype(jnp.bfloat16)
        wb = w_ref[:, :].astype(jnp.bfloat16)
        logits_t = lax.dot_general(
            wb,
            xb,
            (((0,), (1,)), ((), ())),
            preferred_element_type=jnp.float32,
        )

        s = jnp.sum(jnp.exp(logits_t), axis=0, keepdims=True)

        l_dma.wait()
        vids = lax.broadcasted_iota(jnp.int32, (V_LOCAL, T), 0) + my * V_LOCAL
        mask = vids == labels_ref[:].reshape(1, T)
        ll = jnp.sum(jnp.where(mask, logits_t, 0.0), axis=0, keepdims=True)

        comm_ref[0, :, :] = jnp.concatenate([s, ll], axis=0)

        pl.semaphore_wait(barrier, N_DEV - 1)

        sends = []
        for d in (4, 3, 5, 2, 6, 1, 7):
            rdma = pltpu.make_async_remote_copy(
                src_ref=comm_ref.at[0],
                dst_ref=comm_ref.at[d],
                send_sem=send_sems.at[d],
                recv_sem=recv_sems.at[d],
                device_id=((my + d) % N_DEV,),
                device_id_type=pl.DeviceIdType.MESH,
            )
            rdma.start()
            sends.append(rdma)

        for rdma in sends:
            rdma.wait_recv()
        for rdma in sends:
            rdma.wait_send()

        c = comm_ref[:, :, :]
        s_g = jnp.sum(c[:, 0:1, :], axis=0)
        ll_g = jnp.sum(c[:, 1:2, :], axis=0)
        out_ref[:] = (jnp.log(s_g) - ll_g).reshape(T)

    return pl.pallas_call(
        body,
        out_shape=jax.ShapeDtypeStruct((T,), jnp.float32),
        in_specs=[
            pl.BlockSpec(memory_space=pltpu.MemorySpace.HBM),
            pl.BlockSpec(memory_space=pltpu.VMEM),
            pl.BlockSpec(memory_space=pltpu.MemorySpace.HBM),
        ],
        out_specs=pl.BlockSpec(memory_space=pltpu.VMEM),
        scratch_shapes=[
            pltpu.VMEM((T, D), jnp.float32),
            pltpu.VMEM((T,), jnp.int32),
            pltpu.VMEM((N_DEV, 2, T), jnp.float32),
            pltpu.SemaphoreType.DMA((2,)),
            pltpu.SemaphoreType.DMA((N_DEV,)),
            pltpu.SemaphoreType.DMA((N_DEV,)),
        ],
        compiler_params=pltpu.CompilerParams(collective_id=0),
    )(x, W, labels)


# device time: 13601 ns/iter; 1.2878x vs baseline; 1.0292x over previous
import jax
import jax.numpy as jnp
from jax import lax
from jax.experimental import pallas as pl
from jax.experimental.pallas import tpu as pltpu

N_DEV = 8
T = 256
D = 512
V_LOCAL = 4096


def kernel(x, W, labels):
    def body(
        x_ref,
        w_ref,
        labels_ref,
        out_ref,
        comm_ref,
        send_sems,
        recv_sems,
    ):
        my = lax.axis_index("i")

        barrier = pltpu.get_barrier_semaphore()
        for d in range(1, N_DEV):
            pl.semaphore_signal(
                barrier,
                inc=1,
                device_id=((my + d) % N_DEV,),
                device_id_type=pl.DeviceIdType.MESH,
            )

        xb = x_ref[:, :].astype(jnp.bfloat16)
        wb = w_ref[:, :].astype(jnp.bfloat16)
        logits_t = lax.dot_general(
            wb,
            xb,
            (((0,), (1,)), ((), ())),
            preferred_element_type=jnp.float32,
        )

        s = jnp.sum(jnp.exp(logits_t), axis=0, keepdims=True)

        vids = lax.broadcasted_iota(jnp.int32, (V_LOCAL, T), 0) + my * V_LOCAL
        mask = vids == labels_ref[:].reshape(1, T)
        ll = jnp.sum(jnp.where(mask, logits_t, 0.0), axis=0, keepdims=True)

        comm_ref[0, :, :] = jnp.concatenate([s, ll], axis=0)

        pl.semaphore_wait(barrier, N_DEV - 1)

        sends = []
        for d in (4, 3, 5, 2, 6, 1, 7):
            rdma = pltpu.make_async_remote_copy(
                src_ref=comm_ref.at[0],
                dst_ref=comm_ref.at[d],
                send_sem=send_sems.at[d],
                recv_sem=recv_sems.at[d],
                device_id=((my + d) % N_DEV,),
                device_id_type=pl.DeviceIdType.MESH,
            )
            rdma.start()
            sends.append(rdma)

        for rdma in sends:
            rdma.wait_recv()
        for rdma in sends:
            rdma.wait_send()

        c = comm_ref[:, :, :]
        s_g = jnp.sum(c[:, 0:1, :], axis=0)
        ll_g = jnp.sum(c[:, 1:2, :], axis=0)
        out_ref[:] = (jnp.log(s_g) - ll_g).reshape(T)

    return pl.pallas_call(
        body,
        out_shape=jax.ShapeDtypeStruct((T,), jnp.float32),
        in_specs=[
            pl.BlockSpec(memory_space=pltpu.VMEM),
            pl.BlockSpec(memory_space=pltpu.VMEM),
            pl.BlockSpec(memory_space=pltpu.VMEM),
        ],
        out_specs=pl.BlockSpec(memory_space=pltpu.VMEM),
        scratch_shapes=[
            pltpu.VMEM((N_DEV, 2, T), jnp.float32),
            pltpu.SemaphoreType.DMA((N_DEV,)),
            pltpu.SemaphoreType.DMA((N_DEV,)),
        ],
        compiler_params=pltpu.CompilerParams(collective_id=0),
    )(x, W, labels)
